# Initial kernel scaffold; baseline (speedup 1.0000x reference)
import jax
import jax.numpy as jnp
from jax.experimental import pallas as pl


def kernel(x, edge_index, batch, W1, b1, g1, bt1, W2, b2, g2, bt2, W3, b3):
    raise NotImplementedError("write your pallas kernel here")


# speedup vs baseline: 15.9726x; 15.9726x over previous
import functools

import jax
import jax.numpy as jnp
from jax import lax
from jax.experimental import pallas as pl
from jax.experimental.pallas import tpu as pltpu
import jax.experimental.pallas.tpu_sc as plsc

N = 10000
E = 320000
F_IN = 128
H = 64
EMB = 32
G = 16
EPS = 1e-5

NC = 2
NS = 16
NW = NC * NS
CHUNK = 128
NCH = -(-E // (NW * CHUNK))
EPT = NCH * CHUNK
E_PAD = EPT * NW
RPT = 632
N_ACC = RPT * NS

RB = 1000
GRID = N // RB


def _sc_mesh():
    return plsc.VectorSubcoreMesh(core_axis_name="c", subcore_axis_name="s")


def _sc_spmm(hp, src3, dst3, zrows, d):
    @functools.partial(
        pl.kernel,
        mesh=_sc_mesh(),
        compiler_params=pltpu.CompilerParams(use_tc_tiling_on_sc=False),
        out_type=jax.ShapeDtypeStruct((NC, N_ACC, d), jnp.float32),
        scratch_types=[
            pltpu.VMEM((NCH, CHUNK), jnp.int32),
            pltpu.VMEM((NCH, CHUNK), jnp.int32),
            pltpu.VMEM((CHUNK, d), jnp.float32),
            pltpu.VMEM_SHARED((N_ACC, d), jnp.float32),
            pltpu.SemaphoreType.DMA,
        ],
    )
    def k(hp_hbm, src_hbm, dst_hbm, z_hbm, out_hbm, src_v, dst_v, rows_v, acc, sem):
        c = lax.axis_index("c")
        s = lax.axis_index("s")
        wid = c * NS + s
        pltpu.sync_copy(z_hbm, acc.at[pl.ds(s * RPT, RPT)])
        pltpu.sync_copy(src_hbm.at[wid], src_v)
        pltpu.sync_copy(dst_hbm.at[wid], dst_v)
        plsc.subcore_barrier()

        def body(i, carry):
            pltpu.async_copy(hp_hbm.at[src_v.at[i]], rows_v, sem).wait()
            pltpu.sync_copy(rows_v, acc.at[dst_v.at[i]], add=True)
            return carry

        lax.fori_loop(0, NCH, body, 0)
        plsc.subcore_barrier()
        pltpu.sync_copy(acc.at[pl.ds(s * RPT, RPT)],
                        out_hbm.at[c, pl.ds(s * RPT, RPT)])

    return k(hp, src3, dst3, zrows)


def _sc_degree(dst3, ones_rows, zrows):
    @functools.partial(
        pl.kernel,
        mesh=_sc_mesh(),
        compiler_params=pltpu.CompilerParams(use_tc_tiling_on_sc=False),
        out_type=jax.ShapeDtypeStruct((NC, N_ACC, 16), jnp.float32),
        scratch_types=[
            pltpu.VMEM((NCH, CHUNK), jnp.int32),
            pltpu.VMEM((CHUNK, 16), jnp.float32),
            pltpu.VMEM_SHARED((N_ACC, 16), jnp.float32),
        ],
    )
    def k(dst_hbm, ones_hbm, z_hbm, out_hbm, dst_v, ones_v, acc):
        c = lax.axis_index("c")
        s = lax.axis_index("s")
        wid = c * NS + s
        pltpu.sync_copy(z_hbm, acc.at[pl.ds(s * RPT, RPT)])
        pltpu.sync_copy(dst_hbm.at[wid], dst_v)
        pltpu.sync_copy(ones_hbm, ones_v)
        plsc.subcore_barrier()

        def body(i, carry):
            pltpu.sync_copy(ones_v, acc.at[dst_v.at[i]], add=True)
            return carry

        lax.fori_loop(0, NCH, body, 0)
        plsc.subcore_barrier()
        pltpu.sync_copy(acc.at[pl.ds(s * RPT, RPT)],
                        out_hbm.at[c, pl.ds(s * RPT, RPT)])

    return k(dst3, ones_rows, zrows)


def _mm1(dp, x, w):
    def body(dp_ref, x_ref, w_ref, hp_ref, dinv_ref):
        deg = dp_ref[0, :, 0:1] + dp_ref[1, :, 0:1] + 1.0
        dinv = lax.rsqrt(deg)
        h = jnp.dot(x_ref[...], w_ref[...], preferred_element_type=jnp.float32)
        hp_ref[...] = h * dinv
        dinv_ref[...] = dinv

    return pl.pallas_call(
        body,
        grid=(GRID,),
        in_specs=[
            pl.BlockSpec((NC, RB, 16), lambda i: (0, i, 0)),
            pl.BlockSpec((RB, F_IN), lambda i: (i, 0)),
            pl.BlockSpec((F_IN, H), lambda i: (0, 0)),
        ],
        out_specs=[
            pl.BlockSpec((RB, H), lambda i: (i, 0)),
            pl.BlockSpec((RB, 1), lambda i: (i, 0)),
        ],
        out_shape=[
            jax.ShapeDtypeStruct((N, H), jnp.float32),
            jax.ShapeDtypeStruct((N, 1), jnp.float32),
        ],
    )(dp, x, w)


def _post(sp, hp, dinv, b, d):
    def body(s_ref, hp_ref, dinv_ref, b_ref, z_ref, sum_ref, ssq_ref):
        i = pl.program_id(0)
        z = dinv_ref[...] * (s_ref[0] + s_ref[1] + hp_ref[...]) + b_ref[...]
        z_ref[...] = z

        @pl.when(i == 0)
        def _():
            sum_ref[...] = jnp.zeros_like(sum_ref)
            ssq_ref[...] = jnp.zeros_like(ssq_ref)

        sum_ref[...] += jnp.sum(z, axis=0, keepdims=True)
        ssq_ref[...] += jnp.sum(z * z, axis=0, keepdims=True)

    return pl.pallas_call(
        body,
        grid=(GRID,),
        in_specs=[
            pl.BlockSpec((NC, RB, d), lambda i: (0, i, 0)),
            pl.BlockSpec((RB, d), lambda i: (i, 0)),
            pl.BlockSpec((RB, 1), lambda i: (i, 0)),
            pl.BlockSpec((1, d), lambda i: (0, 0)),
        ],
        out_specs=[
            pl.BlockSpec((RB, d), lambda i: (i, 0)),
            pl.BlockSpec((1, d), lambda i: (0, 0)),
            pl.BlockSpec((1, d), lambda i: (0, 0)),
        ],
        out_shape=[
            jax.ShapeDtypeStruct((N, d), jnp.float32),
            jax.ShapeDtypeStruct((1, d), jnp.float32),
            jax.ShapeDtypeStruct((1, d), jnp.float32),
        ],
    )(sp, hp, dinv, b)


def _mm_bn(z, ssum, ssq, g, bt, dinv, w, d_in, d_out):
    def body(z_ref, sum_ref, ssq_ref, g_ref, bt_ref, dinv_ref, w_ref, o_ref):
        mean = sum_ref[...] * (1.0 / N)
        var = ssq_ref[...] * (1.0 / N) - mean * mean
        zn = (z_ref[...] - mean) * lax.rsqrt(var + EPS) * g_ref[...] + bt_ref[...]
        r = jnp.maximum(zn, 0.0)
        o_ref[...] = jnp.dot(r, w_ref[...],
                             preferred_element_type=jnp.float32) * dinv_ref[...]

    return pl.pallas_call(
        body,
        grid=(GRID,),
        in_specs=[
            pl.BlockSpec((RB, d_in), lambda i: (i, 0)),
            pl.BlockSpec((1, d_in), lambda i: (0, 0)),
            pl.BlockSpec((1, d_in), lambda i: (0, 0)),
            pl.BlockSpec((1, d_in), lambda i: (0, 0)),
            pl.BlockSpec((1, d_in), lambda i: (0, 0)),
            pl.BlockSpec((RB, 1), lambda i: (i, 0)),
            pl.BlockSpec((d_in, d_out), lambda i: (0, 0)),
        ],
        out_specs=pl.BlockSpec((RB, d_out), lambda i: (i, 0)),
        out_shape=jax.ShapeDtypeStruct((N, d_out), jnp.float32),
    )(z, ssum, ssq, g, bt, dinv, w)


def _final(sp, hp, dinv, b, batch3):
    def body(s_ref, hp_ref, dinv_ref, b_ref, bt_ref, mean_ref, max_ref, cnt_ref):
        i = pl.program_id(0)
        z = dinv_ref[...] * (s_ref[0] + s_ref[1] + hp_ref[...]) + b_ref[...]
        bt = jnp.reshape(bt_ref[0, 0, :], (RB, 1))
        gid = lax.broadcasted_iota(jnp.int32, (RB, G), 1)
        oh = bt == gid
        ohf = oh.astype(jnp.float32)

        @pl.when(i == 0)
        def _():
            mean_ref[...] = jnp.zeros_like(mean_ref)
            cnt_ref[...] = jnp.zeros_like(cnt_ref)
            max_ref[...] = jnp.full_like(max_ref, -jnp.inf)

        mean_ref[...] += lax.dot_general(
            ohf, z, (((0,), (0,)), ((), ())), preferred_element_type=jnp.float32)
        cnt_ref[...] += jnp.sum(ohf, axis=0, keepdims=True)
        neg = jnp.float32(-jnp.inf)
        upd = jnp.concatenate(
            [jnp.max(jnp.where(oh[:, g:g + 1], z, neg), axis=0, keepdims=True)
             for g in range(G)], axis=0)
        max_ref[...] = jnp.maximum(max_ref[...], upd)

        @pl.when(i == GRID - 1)
        def _():
            cnt = jnp.maximum(cnt_ref[...], 1.0)
            mean_ref[...] = mean_ref[...] / jnp.reshape(cnt, (G, 1))

    return pl.pallas_call(
        body,
        grid=(GRID,),
        in_specs=[
            pl.BlockSpec((NC, RB, EMB), lambda i: (0, i, 0)),
            pl.BlockSpec((RB, EMB), lambda i: (i, 0)),
            pl.BlockSpec((RB, 1), lambda i: (i, 0)),
            pl.BlockSpec((1, EMB), lambda i: (0, 0)),
            pl.BlockSpec((1, 1, RB), lambda i: (i, 0, 0)),
        ],
        out_specs=[
            pl.BlockSpec((G, EMB), lambda i: (0, 0)),
            pl.BlockSpec((G, EMB), lambda i: (0, 0)),
            pl.BlockSpec((1, G), lambda i: (0, 0)),
        ],
        out_shape=[
            jax.ShapeDtypeStruct((G, EMB), jnp.float32),
            jax.ShapeDtypeStruct((G, EMB), jnp.float32),
            jax.ShapeDtypeStruct((1, G), jnp.float32),
        ],
    )(sp, hp, dinv, b, batch3)


def kernel(x, edge_index, batch, W1, b1, g1, bt1, W2, b2, g2, bt2, W3, b3):
    pad = E_PAD - E
    src3 = jnp.concatenate(
        [edge_index[0], jnp.zeros((pad,), jnp.int32)]).reshape(NW, NCH, CHUNK)
    dst3 = jnp.concatenate(
        [edge_index[1], jnp.full((pad,), N, jnp.int32)]).reshape(NW, NCH, CHUNK)
    z16 = jnp.zeros((RPT, 16), jnp.float32)
    zH = jnp.zeros((RPT, H), jnp.float32)
    zE = jnp.zeros((RPT, EMB), jnp.float32)
    ones16 = jnp.ones((CHUNK, 16), jnp.float32)
    batch3 = batch.reshape(GRID, 1, RB)
    b1r, b2r, b3r = b1.reshape(1, H), b2.reshape(1, H), b3.reshape(1, EMB)
    g1r, g2r = g1.reshape(1, H), g2.reshape(1, H)
    bt1r, bt2r = bt1.reshape(1, H), bt2.reshape(1, H)

    dp = _sc_degree(dst3, ones16, z16)
    hp1, dinv = _mm1(dp[:, :N, :], x, W1)
    s1 = _sc_spmm(hp1, src3, dst3, zH, H)
    z1, sm1, sq1 = _post(s1, hp1, dinv, b1r, H)
    hp2 = _mm_bn(z1, sm1, sq1, g1r, bt1r, dinv, W2, H, H)
    s2 = _sc_spmm(hp2, src3, dst3, zH, H)
    z2, sm2, sq2 = _post(s2, hp2, dinv, b2r, H)
    hp3 = _mm_bn(z2, sm2, sq2, g2r, bt2r, dinv, W3, H, EMB)
    s3 = _sc_spmm(hp3, src3, dst3, zE, EMB)
    mean, mx, _ = _final(s3, hp3, dinv, b3r, batch3)
    return jnp.concatenate([mean, mx], axis=1)


# speedup vs baseline: 17.6385x; 1.1043x over previous
import functools

import jax
import jax.numpy as jnp
from jax import lax
from jax.experimental import pallas as pl
from jax.experimental.pallas import tpu as pltpu
import jax.experimental.pallas.tpu_sc as plsc

N = 10000
E = 320000
F_IN = 128
H = 64
EMB = 32
G = 16
EPS = 1e-5

NC = 2
NS = 16
NW = NC * NS
CHUNK = 128
NB = 8
NCH = 80
EPT = NCH * CHUNK
E_PAD = EPT * NW
RPT = 632
N_ACC = RPT * NS

RB = 1000
GRID = N // RB


def _sc_mesh():
    return plsc.VectorSubcoreMesh(core_axis_name="c", subcore_axis_name="s")


def _sc_spmm(hp, src3, dst3, zrows, d):
    @functools.partial(
        pl.kernel,
        mesh=_sc_mesh(),
        compiler_params=pltpu.CompilerParams(use_tc_tiling_on_sc=False),
        out_type=jax.ShapeDtypeStruct((NC, N_ACC, d), jnp.float32),
        scratch_types=[
            pltpu.VMEM((NCH, CHUNK), jnp.int32),
            pltpu.VMEM((NCH, CHUNK), jnp.int32),
            pltpu.VMEM((NB, CHUNK, d), jnp.float32),
            pltpu.VMEM_SHARED((N_ACC, d), jnp.float32),
            pltpu.SemaphoreType.DMA((NB,)),
        ],
    )
    def k(hp_hbm, src_hbm, dst_hbm, z_hbm, out_hbm, src_v, dst_v, bufs, acc, gsem):
        c = lax.axis_index("c")
        s = lax.axis_index("s")
        wid = c * NS + s
        pltpu.sync_copy(z_hbm, acc.at[pl.ds(s * RPT, RPT)])
        pltpu.sync_copy(src_hbm.at[wid], src_v)
        pltpu.sync_copy(dst_hbm.at[wid], dst_v)
        plsc.subcore_barrier()

        for b in range(NB):
            pltpu.async_copy(hp_hbm.at[src_v.at[b]], bufs.at[b], gsem.at[b])

        def body(p, carry):
            for b in range(NB):
                i = p * NB + b
                pltpu.make_async_copy(
                    hp_hbm.at[src_v.at[i]], bufs.at[b], gsem.at[b]).wait()
                pltpu.sync_copy(bufs.at[b], acc.at[dst_v.at[i]], add=True)

                @pl.when(p < NCH // NB - 1)
                def _():
                    pltpu.async_copy(
                        hp_hbm.at[src_v.at[i + NB]], bufs.at[b], gsem.at[b])
            return carry

        lax.fori_loop(0, NCH // NB, body, 0)
        plsc.subcore_barrier()
        pltpu.sync_copy(acc.at[pl.ds(s * RPT, RPT)],
                        out_hbm.at[c, pl.ds(s * RPT, RPT)])

    return k(hp, src3, dst3, zrows)


def _sc_degree(dst3, ones_rows, zrows):
    @functools.partial(
        pl.kernel,
        mesh=_sc_mesh(),
        compiler_params=pltpu.CompilerParams(use_tc_tiling_on_sc=False),
        out_type=jax.ShapeDtypeStruct((NC, N_ACC, 16), jnp.float32),
        scratch_types=[
            pltpu.VMEM((NCH, CHUNK), jnp.int32),
            pltpu.VMEM((CHUNK, 16), jnp.float32),
            pltpu.VMEM_SHARED((N_ACC, 16), jnp.float32),
        ],
    )
    def k(dst_hbm, ones_hbm, z_hbm, out_hbm, dst_v, ones_v, acc):
        c = lax.axis_index("c")
        s = lax.axis_index("s")
        wid = c * NS + s
        pltpu.sync_copy(z_hbm, acc.at[pl.ds(s * RPT, RPT)])
        pltpu.sync_copy(dst_hbm.at[wid], dst_v)
        pltpu.sync_copy(ones_hbm, ones_v)
        plsc.subcore_barrier()

        def body(i, carry):
            pltpu.sync_copy(ones_v, acc.at[dst_v.at[i]], add=True)
            return carry

        lax.fori_loop(0, NCH, body, 0)
        plsc.subcore_barrier()
        pltpu.sync_copy(acc.at[pl.ds(s * RPT, RPT)],
                        out_hbm.at[c, pl.ds(s * RPT, RPT)])

    return k(dst3, ones_rows, zrows)


def _mm1(dp, x, w):
    def body(dp_ref, x_ref, w_ref, hp_ref, dinv_ref):
        deg = dp_ref[0, :, 0:1] + dp_ref[1, :, 0:1] + 1.0
        dinv = lax.rsqrt(deg)
        h = jnp.dot(x_ref[...], w_ref[...], preferred_element_type=jnp.float32)
        hp_ref[...] = h * dinv
        dinv_ref[...] = dinv

    return pl.pallas_call(
        body,
        grid=(GRID,),
        in_specs=[
            pl.BlockSpec((NC, RB, 16), lambda i: (0, i, 0)),
            pl.BlockSpec((RB, F_IN), lambda i: (i, 0)),
            pl.BlockSpec((F_IN, H), lambda i: (0, 0)),
        ],
        out_specs=[
            pl.BlockSpec((RB, H), lambda i: (i, 0)),
            pl.BlockSpec((RB, 1), lambda i: (i, 0)),
        ],
        out_shape=[
            jax.ShapeDtypeStruct((N, H), jnp.float32),
            jax.ShapeDtypeStruct((N, 1), jnp.float32),
        ],
    )(dp, x, w)


def _post(sp, hp, dinv, b, d):
    def body(s_ref, hp_ref, dinv_ref, b_ref, z_ref, sum_ref, ssq_ref):
        i = pl.program_id(0)
        z = dinv_ref[...] * (s_ref[0] + s_ref[1] + hp_ref[...]) + b_ref[...]
        z_ref[...] = z

        @pl.when(i == 0)
        def _():
            sum_ref[...] = jnp.zeros_like(sum_ref)
            ssq_ref[...] = jnp.zeros_like(ssq_ref)

        sum_ref[...] += jnp.sum(z, axis=0, keepdims=True)
        ssq_ref[...] += jnp.sum(z * z, axis=0, keepdims=True)

    return pl.pallas_call(
        body,
        grid=(GRID,),
        in_specs=[
            pl.BlockSpec((NC, RB, d), lambda i: (0, i, 0)),
            pl.BlockSpec((RB, d), lambda i: (i, 0)),
            pl.BlockSpec((RB, 1), lambda i: (i, 0)),
            pl.BlockSpec((1, d), lambda i: (0, 0)),
        ],
        out_specs=[
            pl.BlockSpec((RB, d), lambda i: (i, 0)),
            pl.BlockSpec((1, d), lambda i: (0, 0)),
            pl.BlockSpec((1, d), lambda i: (0, 0)),
        ],
        out_shape=[
            jax.ShapeDtypeStruct((N, d), jnp.float32),
            jax.ShapeDtypeStruct((1, d), jnp.float32),
            jax.ShapeDtypeStruct((1, d), jnp.float32),
        ],
    )(sp, hp, dinv, b)


def _mm_bn(z, ssum, ssq, g, bt, dinv, w, d_in, d_out):
    def body(z_ref, sum_ref, ssq_ref, g_ref, bt_ref, dinv_ref, w_ref, o_ref):
        mean = sum_ref[...] * (1.0 / N)
        var = ssq_ref[...] * (1.0 / N) - mean * mean
        zn = (z_ref[...] - mean) * lax.rsqrt(var + EPS) * g_ref[...] + bt_ref[...]
        r = jnp.maximum(zn, 0.0)
        o_ref[...] = jnp.dot(r, w_ref[...],
                             preferred_element_type=jnp.float32) * dinv_ref[...]

    return pl.pallas_call(
        body,
        grid=(GRID,),
        in_specs=[
            pl.BlockSpec((RB, d_in), lambda i: (i, 0)),
            pl.BlockSpec((1, d_in), lambda i: (0, 0)),
            pl.BlockSpec((1, d_in), lambda i: (0, 0)),
            pl.BlockSpec((1, d_in), lambda i: (0, 0)),
            pl.BlockSpec((1, d_in), lambda i: (0, 0)),
            pl.BlockSpec((RB, 1), lambda i: (i, 0)),
            pl.BlockSpec((d_in, d_out), lambda i: (0, 0)),
        ],
        out_specs=pl.BlockSpec((RB, d_out), lambda i: (i, 0)),
        out_shape=jax.ShapeDtypeStruct((N, d_out), jnp.float32),
    )(z, ssum, ssq, g, bt, dinv, w)


def _final(sp, hp, dinv, b, batch3):
    def body(s_ref, hp_ref, dinv_ref, b_ref, bt_ref, mean_ref, max_ref, cnt_ref):
        i = pl.program_id(0)
        z = dinv_ref[...] * (s_ref[0] + s_ref[1] + hp_ref[...]) + b_ref[...]
        bt = jnp.reshape(bt_ref[0, 0, :], (RB, 1))
        gid = lax.broadcasted_iota(jnp.int32, (RB, G), 1)
        oh = bt == gid
        ohf = oh.astype(jnp.float32)

        @pl.when(i == 0)
        def _():
            mean_ref[...] = jnp.zeros_like(mean_ref)
            cnt_ref[...] = jnp.zeros_like(cnt_ref)
            max_ref[...] = jnp.full_like(max_ref, -jnp.inf)

        mean_ref[...] += lax.dot_general(
            ohf, z, (((0,), (0,)), ((), ())), preferred_element_type=jnp.float32)
        cnt_ref[...] += jnp.sum(ohf, axis=0, keepdims=True)
        neg = jnp.float32(-jnp.inf)
        upd = jnp.concatenate(
            [jnp.max(jnp.where(oh[:, g:g + 1], z, neg), axis=0, keepdims=True)
             for g in range(G)], axis=0)
        max_ref[...] = jnp.maximum(max_ref[...], upd)

        @pl.when(i == GRID - 1)
        def _():
            cnt = jnp.maximum(cnt_ref[...], 1.0)
            mean_ref[...] = mean_ref[...] / jnp.reshape(cnt, (G, 1))

    return pl.pallas_call(
        body,
        grid=(GRID,),
        in_specs=[
            pl.BlockSpec((NC, RB, EMB), lambda i: (0, i, 0)),
            pl.BlockSpec((RB, EMB), lambda i: (i, 0)),
            pl.BlockSpec((RB, 1), lambda i: (i, 0)),
            pl.BlockSpec((1, EMB), lambda i: (0, 0)),
            pl.BlockSpec((1, 1, RB), lambda i: (i, 0, 0)),
        ],
        out_specs=[
            pl.BlockSpec((G, EMB), lambda i: (0, 0)),
            pl.BlockSpec((G, EMB), lambda i: (0, 0)),
            pl.BlockSpec((1, G), lambda i: (0, 0)),
        ],
        out_shape=[
            jax.ShapeDtypeStruct((G, EMB), jnp.float32),
            jax.ShapeDtypeStruct((G, EMB), jnp.float32),
            jax.ShapeDtypeStruct((1, G), jnp.float32),
        ],
    )(sp, hp, dinv, b, batch3)


def kernel(x, edge_index, batch, W1, b1, g1, bt1, W2, b2, g2, bt2, W3, b3):
    pad = E_PAD - E
    src3 = jnp.concatenate(
        [edge_index[0], jnp.zeros((pad,), jnp.int32)]).reshape(NW, NCH, CHUNK)
    dst3 = jnp.concatenate(
        [edge_index[1],
         N + jnp.arange(pad, dtype=jnp.int32) % (N_ACC - N)]
    ).reshape(NW, NCH, CHUNK)
    z16 = jnp.zeros((RPT, 16), jnp.float32)
    zH = jnp.zeros((RPT, H), jnp.float32)
    zE = jnp.zeros((RPT, EMB), jnp.float32)
    ones16 = jnp.ones((CHUNK, 16), jnp.float32)
    batch3 = batch.reshape(GRID, 1, RB)
    b1r, b2r, b3r = b1.reshape(1, H), b2.reshape(1, H), b3.reshape(1, EMB)
    g1r, g2r = g1.reshape(1, H), g2.reshape(1, H)
    bt1r, bt2r = bt1.reshape(1, H), bt2.reshape(1, H)

    dp = _sc_degree(dst3, ones16, z16)
    hp1, dinv = _mm1(dp[:, :N, :], x, W1)
    s1 = _sc_spmm(hp1, src3, dst3, zH, H)
    z1, sm1, sq1 = _post(s1, hp1, dinv, b1r, H)
    hp2 = _mm_bn(z1, sm1, sq1, g1r, bt1r, dinv, W2, H, H)
    s2 = _sc_spmm(hp2, src3, dst3, zH, H)
    z2, sm2, sq2 = _post(s2, hp2, dinv, b2r, H)
    hp3 = _mm_bn(z2, sm2, sq2, g2r, bt2r, dinv, W3, H, EMB)
    s3 = _sc_spmm(hp3, src3, dst3, zE, EMB)
    mean, mx, _ = _final(s3, hp3, dinv, b3r, batch3)
    return jnp.concatenate([mean, mx], axis=1)
